# Initial kernel scaffold; baseline (speedup 1.0000x reference)
import jax
import jax.numpy as jnp
from jax.experimental import pallas as pl


def kernel(x, edge_index, l1_W1, l1_b1, l1_g1, l1_bt1, l1_W2, l1_b2, l1_g2, l1_bt2, l1_eps, l2_W1, l2_b1, l2_g1, l2_bt1, l2_W2, l2_b2, l2_g2, l2_bt2, l2_eps, l3_W1, l3_b1, l3_g1, l3_bt1, l3_W2, l3_b2, l3_g2, l3_bt2, l3_eps):
    raise NotImplementedError("write your pallas kernel here")


# speedup vs baseline: 5.0218x; 5.0218x over previous
import functools

import jax
import jax.numpy as jnp
from jax import lax
from jax.experimental import pallas as pl
from jax.experimental.pallas import tpu as pltpu
from jax.experimental.pallas import tpu_sc as plsc

_NC = 2
_NS = 16
_CH = 80
_NP = 10240
_ZR = 16
_W = 128
_PB = 125


@functools.lru_cache(maxsize=None)
def _make_sc_aggregate(e, feature_split):
    rows_per_tile = _NP // _NS
    nworkers = _NS if feature_split else _NC * _NS
    chunks_per_tile = e // _CH // nworkers
    nphases = chunks_per_tile // _PB
    assert rows_per_tile % _ZR == 0 and chunks_per_tile % _PB == 0

    mesh = plsc.VectorSubcoreMesh(
        core_axis_name="c", subcore_axis_name="s",
        num_cores=_NC, num_subcores=_NS)

    @functools.partial(
        pl.kernel,
        out_type=jax.ShapeDtypeStruct((2 * _NP, _W), jnp.float32),
        mesh=mesh,
        scratch_types=[
            pltpu.VMEM((_PB, _CH), jnp.int32),
            pltpu.VMEM((_PB, _CH), jnp.int32),
            pltpu.VMEM((_CH, _W), jnp.float32),
            pltpu.VMEM((_ZR, _W), jnp.float32),
            pltpu.VMEM_SHARED((_NP, _W), jnp.float32),
            pltpu.SemaphoreType.DMA,
        ],
    )
    def agg_kernel(h_hbm, idx_hbm, out_hbm,
                   src_v, dst_v, rows_v, zero_v, acc_sh, sem):
        c = lax.axis_index("c")
        s = lax.axis_index("s")

        def _zz(i, carry):
            zero_v[i // (_W // 16), pl.ds((i % (_W // 16)) * 16, 16)] = (
                jnp.zeros((16,), jnp.float32))
            return carry
        lax.fori_loop(0, _ZR * (_W // 16), _zz, 0)
        for k in range(rows_per_tile // _ZR):
            pltpu.sync_copy(
                zero_v,
                acc_sh.at[pl.ds(s * rows_per_tile + k * _ZR, _ZR)])

        wid = c * _NS + s
        dstb = s if feature_split else wid
        plsc.subcore_barrier()

        def _edge_chunk(j, carry):
            pltpu.async_copy(h_hbm.at[src_v.at[j]], rows_v, sem).wait()
            pltpu.sync_copy(rows_v, acc_sh.at[dst_v.at[j]], add=True)
            return carry

        for p in range(nphases):
            pltpu.sync_copy(idx_hbm.at[wid * nphases + p], src_v)
            pltpu.sync_copy(
                idx_hbm.at[32 * nphases + dstb * nphases + p], dst_v)
            lax.fori_loop(0, _PB, _edge_chunk, 0)

        plsc.subcore_barrier()
        pltpu.sync_copy(
            acc_sh.at[pl.ds(s * rows_per_tile, rows_per_tile)],
            out_hbm.at[pl.ds(c * _NP + s * rows_per_tile, rows_per_tile)])

    return agg_kernel


def _pack_idx(src_blocks, dst_blocks):
    used = jnp.concatenate([
        src_blocks.reshape(-1, _PB, _CH), dst_blocks.reshape(-1, _PB, _CH)],
        axis=0)
    nblocks = (2 * 1024 * 1024 + 128 * 1024) // (_PB * _CH) + 1
    return jnp.pad(used, ((0, nblocks - used.shape[0]), (0, 0), (0, 0)))


def _mlp_body(n, din, hdim, first, split_out,
              hin_ref, agg_ref, eps_ref, w1_ref, b1_ref, g1_ref, bt1_ref,
              w2_ref, b2_ref, g2_ref, bt2_ref, out_ref):
    one_p_eps = 1.0 + eps_ref[0, 0]
    if first:
        u = (one_p_eps * hin_ref[...]
             + agg_ref[:n, :] + agg_ref[_NP:_NP + n, :])
        z = (jnp.dot(u, w1_ref[...], preferred_element_type=jnp.float32)
             + b1_ref[...])
    else:
        hw = din // 2
        ua = one_p_eps * hin_ref[:n, :] + agg_ref[:n, :]
        ub = (one_p_eps * hin_ref[_NP:_NP + n, :]
              + agg_ref[_NP:_NP + n, :])
        w1 = w1_ref[...]
        z = (jnp.dot(ua, w1[:hw, :], preferred_element_type=jnp.float32)
             + jnp.dot(ub, w1[hw:, :], preferred_element_type=jnp.float32)
             + b1_ref[...])

    def _bn_relu(z, g_ref, bt_ref):
        m = jnp.mean(z, axis=0, keepdims=True)
        v = jnp.mean(z * z, axis=0, keepdims=True) - m * m
        return jnp.maximum(
            g_ref[...] * (z - m) * lax.rsqrt(v + 1e-5) + bt_ref[...], 0.0)

    a = _bn_relu(z, g1_ref, bt1_ref)
    z2 = (jnp.dot(a, w2_ref[...], preferred_element_type=jnp.float32)
          + b2_ref[...])
    y = _bn_relu(z2, g2_ref, bt2_ref)
    if split_out:
        out_ref[:n, :] = y[:, :hdim // 2]
        out_ref[_NP:_NP + n, :] = y[:, hdim // 2:]
    else:
        out_ref[...] = y


@functools.lru_cache(maxsize=None)
def _make_mlp(n, din, hdim, first, split_out):
    out_shape = (2 * _NP, hdim // 2) if split_out else (n, hdim)
    vmem = pl.BlockSpec(memory_space=pltpu.VMEM)
    smem = pl.BlockSpec(memory_space=pltpu.SMEM)
    return pl.pallas_call(
        functools.partial(_mlp_body, n, din, hdim, first, split_out),
        out_shape=jax.ShapeDtypeStruct(out_shape, jnp.float32),
        in_specs=[vmem, vmem, smem] + [vmem] * 8,
        out_specs=vmem,
    )


def _layer(h_split, hin, idx, e, n, din, hdim, first, split_out,
           eps, w1, b1, g1, bt1, w2, b2, g2, bt2):
    agg = _make_sc_aggregate(e, not first)(h_split, idx)
    r = lambda v: v.reshape(1, -1)
    return _make_mlp(n, din, hdim, first, split_out)(
        hin, agg, eps.reshape(1, 1), w1, r(b1), r(g1), r(bt1),
        w2, r(b2), r(g2), r(bt2))


def kernel(x, edge_index,
           l1_W1, l1_b1, l1_g1, l1_bt1, l1_W2, l1_b2, l1_g2, l1_bt2, l1_eps,
           l2_W1, l2_b1, l2_g1, l2_bt1, l2_W2, l2_b2, l2_g2, l2_bt2, l2_eps,
           l3_W1, l3_b1, l3_g1, l3_bt1, l3_W2, l3_b2, l3_g2, l3_bt2, l3_eps):
    n, d = x.shape
    e = edge_index.shape[1]
    hdim = l1_W1.shape[1]
    src = edge_index[0]
    dst = edge_index[1]
    idx_a = _pack_idx(
        src.reshape(2 * _NS, e // _CH // (2 * _NS), _CH),
        dst.reshape(2 * _NS, e // _CH // (2 * _NS), _CH))
    idx_b = _pack_idx(
        jnp.stack([src, src + _NP]).reshape(2 * _NS, e // _CH // _NS, _CH),
        dst.reshape(_NS, e // _CH // _NS, _CH))

    h1 = _layer(x, x, idx_a, e, n, d, hdim, True, True,
                l1_eps, l1_W1, l1_b1, l1_g1, l1_bt1,
                l1_W2, l1_b2, l1_g2, l1_bt2)
    h2 = _layer(h1, h1, idx_b, e, n, hdim, hdim, False, True,
                l2_eps, l2_W1, l2_b1, l2_g1, l2_bt1,
                l2_W2, l2_b2, l2_g2, l2_bt2)
    return _layer(h2, h2, idx_b, e, n, hdim, hdim, False, False,
                  l3_eps, l3_W1, l3_b1, l3_g1, l3_bt1,
                  l3_W2, l3_b2, l3_g2, l3_bt2)
